# baseline (speedup 1.0000x reference)
import functools

import jax
import jax.numpy as jnp
from jax import lax
from jax.experimental import pallas as pl
from jax.experimental.pallas import tpu as pltpu
from jax.experimental.pallas import tpu_sc as plsc

N_PAIRS = 65536
FEAT = 320
HIDDEN = 128
CHUNK = 128


def _sc_gather_body(d_hbm, p_hbm, didx_hbm, pidx_hbm, outd_hbm, outp_hbm,
                    idx_v, rows_v, idx2_v, rows2_v, sem, sem2):
    nc = 2
    wid = lax.axis_index("s") * nc + lax.axis_index("c")
    per_w = N_PAIRS // 32
    n_chunks = per_w // CHUNK
    base = wid * per_w

    def body(i, carry):
        off = base + i * CHUNK
        pltpu.sync_copy(didx_hbm.at[pl.ds(off, CHUNK)], idx_v)
        cp_d = pltpu.async_copy(d_hbm.at[idx_v], rows_v, sem)
        pltpu.sync_copy(pidx_hbm.at[pl.ds(off, CHUNK)], idx2_v)
        cp_p = pltpu.async_copy(p_hbm.at[idx2_v], rows2_v, sem2)
        cp_d.wait()
        pltpu.sync_copy(rows_v, outd_hbm.at[pl.ds(off, CHUNK)])
        cp_p.wait()
        pltpu.sync_copy(rows2_v, outp_hbm.at[pl.ds(off, CHUNK)])
        return carry

    lax.fori_loop(0, n_chunks, body, 0)


@functools.partial(jax.jit, static_argnums=())
def _sc_gather(d, p, didx, pidx):
    mesh = plsc.VectorSubcoreMesh(core_axis_name="c", subcore_axis_name="s")
    return pl.kernel(
        _sc_gather_body,
        out_type=(
            jax.ShapeDtypeStruct((N_PAIRS, FEAT), jnp.float32),
            jax.ShapeDtypeStruct((N_PAIRS, FEAT), jnp.float32),
        ),
        mesh=mesh,
        scratch_types=[
            pltpu.VMEM((CHUNK,), jnp.int32),
            pltpu.VMEM((CHUNK, FEAT), jnp.float32),
            pltpu.VMEM((CHUNK,), jnp.int32),
            pltpu.VMEM((CHUNK, FEAT), jnp.float32),
            pltpu.SemaphoreType.DMA,
            pltpu.SemaphoreType.DMA,
        ],
        compiler_params=pltpu.CompilerParams(use_tc_tiling_on_sc=False),
    )(d, p, didx, pidx)


def _mlp_body(xd_ref, xp_ref, w1a_ref, w1b_ref, b1_ref, w2_ref, b2_ref, o_ref):
    acc = jnp.dot(xd_ref[...], w1a_ref[...], preferred_element_type=jnp.float32)
    acc += jnp.dot(xp_ref[...], w1b_ref[...], preferred_element_type=jnp.float32)
    h = jnp.tanh(acc + b1_ref[...])
    logits = jnp.dot(h, w2_ref[...], preferred_element_type=jnp.float32)
    logits += b2_ref[...]
    m = jnp.max(logits, axis=1, keepdims=True)
    lse = m + jnp.log(jnp.sum(jnp.exp(logits - m), axis=1, keepdims=True))
    o_ref[...] = logits - lse


def _tc_mlp(xd, xp, w1a, w1b, b1, w2, b2):
    blk = 2048
    grid = (N_PAIRS // blk,)
    return pl.pallas_call(
        _mlp_body,
        grid=grid,
        in_specs=[
            pl.BlockSpec((blk, FEAT), lambda i: (i, 0)),
            pl.BlockSpec((blk, FEAT), lambda i: (i, 0)),
            pl.BlockSpec((FEAT, HIDDEN), lambda i: (0, 0)),
            pl.BlockSpec((FEAT, HIDDEN), lambda i: (0, 0)),
            pl.BlockSpec((1, HIDDEN), lambda i: (0, 0)),
            pl.BlockSpec((HIDDEN, 2), lambda i: (0, 0)),
            pl.BlockSpec((1, 2), lambda i: (0, 0)),
        ],
        out_specs=pl.BlockSpec((blk, 2), lambda i: (i, 0)),
        out_shape=jax.ShapeDtypeStruct((N_PAIRS, 2), jnp.float32),
        compiler_params=pltpu.CompilerParams(
            dimension_semantics=("arbitrary",),
        ),
    )(xd, xp, w1a, w1b, b1, w2, b2)


def kernel(graph, dataset_index, iftrain, d, p, W1, b1, W2, b2):
    didx = dataset_index[:, 0].astype(jnp.int32)
    pidx = dataset_index[:, 1].astype(jnp.int32)
    xd, xp = _sc_gather(d, p, didx, pidx)
    w1a = W1[:FEAT]
    w1b = W1[FEAT:]
    return _tc_mlp(xd, xp, w1a, w1b, b1.reshape(1, HIDDEN), W2,
                   b2.reshape(1, 2))


# speedup vs baseline: 2.6602x; 2.6602x over previous
import functools

import jax
import jax.numpy as jnp
from jax import lax
from jax.experimental import pallas as pl
from jax.experimental.pallas import tpu as pltpu
from jax.experimental.pallas import tpu_sc as plsc

N_PAIRS = 65536
N_NODES = 100000
FEAT = 320
HIDDEN = 128
CHUNK = 128


def _embed_body(t_ref, w_ref, o_ref):
    o_ref[...] = jnp.dot(t_ref[...], w_ref[...],
                         preferred_element_type=jnp.float32)


def _tc_embed(table, w):
    blk = 800
    return pl.pallas_call(
        _embed_body,
        grid=(N_NODES // blk,),
        in_specs=[
            pl.BlockSpec((blk, FEAT), lambda i: (i, 0)),
            pl.BlockSpec((FEAT, HIDDEN), lambda i: (0, 0)),
        ],
        out_specs=pl.BlockSpec((blk, HIDDEN), lambda i: (i, 0)),
        out_shape=jax.ShapeDtypeStruct((N_NODES, HIDDEN), jnp.float32),
        compiler_params=pltpu.CompilerParams(
            dimension_semantics=("arbitrary",),
        ),
    )(table, w)


def _sc_gather_body(dp_hbm, pp_hbm, didx_hbm, pidx_hbm, outd_hbm, outp_hbm,
                    idx_v, rows_d0, rows_d1, rows_p0, rows_p1,
                    sem_d0, sem_d1, sem_p0, sem_p1):
    nc = 2
    wid = lax.axis_index("s") * nc + lax.axis_index("c")
    per_w = N_PAIRS // 32
    n_chunks = per_w // CHUNK
    base = wid * per_w

    pltpu.sync_copy(didx_hbm.at[pl.ds(base, per_w)],
                    idx_v.at[pl.ds(0, per_w)])
    pltpu.sync_copy(pidx_hbm.at[pl.ds(base, per_w)],
                    idx_v.at[pl.ds(per_w, per_w)])

    slots = ((rows_d0, rows_p0, sem_d0, sem_p0),
             (rows_d1, rows_p1, sem_d1, sem_p1))

    def start(c, slot):
        rd, rp, sd, sp = slots[slot]
        pltpu.async_copy(dp_hbm.at[idx_v.at[pl.ds(c * CHUNK, CHUNK)]], rd, sd)
        pltpu.async_copy(pp_hbm.at[idx_v.at[pl.ds(per_w + c * CHUNK, CHUNK)]],
                         rp, sp)

    def finish(c, slot):
        rd, rp, sd, sp = slots[slot]
        off = base + c * CHUNK
        pltpu.make_async_copy(dp_hbm.at[pl.ds(0, CHUNK)], rd, sd).wait()
        pltpu.sync_copy(rd, outd_hbm.at[pl.ds(off, CHUNK)])
        pltpu.make_async_copy(pp_hbm.at[pl.ds(0, CHUNK)], rp, sp).wait()
        pltpu.sync_copy(rp, outp_hbm.at[pl.ds(off, CHUNK)])

    start(0, 0)

    def body(j, carry):
        c0 = 2 * j
        start(c0 + 1, 1)
        finish(c0, 0)
        start(c0 + 2, 0)
        finish(c0 + 1, 1)
        return carry

    lax.fori_loop(0, n_chunks // 2 - 1, body, 0)
    c_last = n_chunks - 2
    start(c_last + 1, 1)
    finish(c_last, 0)
    finish(c_last + 1, 1)


def _sc_gather(dp, pp, didx2d, pidx2d):
    mesh = plsc.VectorSubcoreMesh(core_axis_name="c", subcore_axis_name="s")
    n_chunks = N_PAIRS // 32 // CHUNK
    return pl.kernel(
        _sc_gather_body,
        out_type=(
            jax.ShapeDtypeStruct((N_PAIRS, HIDDEN), jnp.float32),
            jax.ShapeDtypeStruct((N_PAIRS, HIDDEN), jnp.float32),
        ),
        mesh=mesh,
        scratch_types=[
            pltpu.VMEM((2 * n_chunks * CHUNK,), jnp.int32),
            pltpu.VMEM((CHUNK, HIDDEN), jnp.float32),
            pltpu.VMEM((CHUNK, HIDDEN), jnp.float32),
            pltpu.VMEM((CHUNK, HIDDEN), jnp.float32),
            pltpu.VMEM((CHUNK, HIDDEN), jnp.float32),
            pltpu.SemaphoreType.DMA,
            pltpu.SemaphoreType.DMA,
            pltpu.SemaphoreType.DMA,
            pltpu.SemaphoreType.DMA,
        ],
    )(dp, pp, didx2d, pidx2d)


def _head_body(xd_ref, xp_ref, b1_ref, w2_ref, b2_ref, o_ref):
    h = jnp.tanh(xd_ref[...] + xp_ref[...] + b1_ref[...])
    logits = jnp.dot(h, w2_ref[...], preferred_element_type=jnp.float32)
    logits += b2_ref[...]
    m = jnp.max(logits, axis=1, keepdims=True)
    lse = m + jnp.log(jnp.sum(jnp.exp(logits - m), axis=1, keepdims=True))
    o_ref[...] = logits - lse


def _tc_head(xd, xp, b1, w2, b2):
    blk = 4096
    return pl.pallas_call(
        _head_body,
        grid=(N_PAIRS // blk,),
        in_specs=[
            pl.BlockSpec((blk, HIDDEN), lambda i: (i, 0)),
            pl.BlockSpec((blk, HIDDEN), lambda i: (i, 0)),
            pl.BlockSpec((1, HIDDEN), lambda i: (0, 0)),
            pl.BlockSpec((HIDDEN, 2), lambda i: (0, 0)),
            pl.BlockSpec((1, 2), lambda i: (0, 0)),
        ],
        out_specs=pl.BlockSpec((blk, 2), lambda i: (i, 0)),
        out_shape=jax.ShapeDtypeStruct((N_PAIRS, 2), jnp.float32),
        compiler_params=pltpu.CompilerParams(
            dimension_semantics=("arbitrary",),
        ),
    )(xd, xp, b1, w2, b2)


def kernel(graph, dataset_index, iftrain, d, p, W1, b1, W2, b2):
    didx = dataset_index[:, 0].astype(jnp.int32)
    pidx = dataset_index[:, 1].astype(jnp.int32)
    dp = _tc_embed(d, W1[:FEAT])
    pp = _tc_embed(p, W1[FEAT:])
    xd, xp = _sc_gather(dp, pp, didx, pidx)
    return _tc_head(xd, xp, b1.reshape(1, HIDDEN), W2, b2.reshape(1, 2))


# speedup vs baseline: 6.7489x; 2.5370x over previous
import jax
import jax.numpy as jnp
from jax import lax
from jax.experimental import pallas as pl
from jax.experimental.pallas import tpu as pltpu
from jax.experimental.pallas import tpu_sc as plsc

N_PAIRS = 65536
N_NODES = 100000
FEAT = 320
HIDDEN = 128
CHUNK = 128


def _embed_body(t_ref, w_ref, o_ref):
    o_ref[...] = lax.dot_general(
        t_ref[...], w_ref[...],
        dimension_numbers=(((0,), (0,)), ((), ())),
        preferred_element_type=jnp.float32)


def _tc_embed(table_t, w):
    blk = 2048
    return pl.pallas_call(
        _embed_body,
        grid=((N_NODES + blk - 1) // blk,),
        in_specs=[
            pl.BlockSpec((FEAT, blk), lambda i: (0, i)),
            pl.BlockSpec((FEAT, HIDDEN), lambda i: (0, 0)),
        ],
        out_specs=pl.BlockSpec((blk, HIDDEN), lambda i: (i, 0)),
        out_shape=jax.ShapeDtypeStruct((N_NODES, HIDDEN), jnp.float32),
        compiler_params=pltpu.CompilerParams(
            dimension_semantics=("arbitrary",),
        ),
    )(table_t, w)


def _sc_gather_body(tab_hbm, idx_hbm, out_hbm,
                    idx_v, rows_0, rows_1, sem_0, sem_1):
    nc = 2
    wid = lax.axis_index("s") * nc + lax.axis_index("c")
    per_w = N_PAIRS // 32
    n_chunks = per_w // CHUNK
    base = wid * per_w

    pltpu.sync_copy(idx_hbm.at[pl.ds(base, per_w)], idx_v)

    slots = ((rows_0, sem_0), (rows_1, sem_1))

    def start(c, slot):
        rows, sem = slots[slot]
        pltpu.async_copy(tab_hbm.at[idx_v.at[pl.ds(c * CHUNK, CHUNK)]],
                         rows, sem)

    def finish(c, slot):
        rows, sem = slots[slot]
        off = base + c * CHUNK
        pltpu.make_async_copy(tab_hbm.at[pl.ds(0, CHUNK)], rows, sem).wait()
        pltpu.sync_copy(rows, out_hbm.at[pl.ds(off, CHUNK)])

    start(0, 0)

    def body(j, carry):
        c0 = 2 * j
        start(c0 + 1, 1)
        finish(c0, 0)
        start(c0 + 2, 0)
        finish(c0 + 1, 1)
        return carry

    lax.fori_loop(0, n_chunks // 2 - 1, body, 0)
    c_last = n_chunks - 2
    start(c_last + 1, 1)
    finish(c_last, 0)
    finish(c_last + 1, 1)


def _sc_gather(tab, idx):
    mesh = plsc.VectorSubcoreMesh(core_axis_name="c", subcore_axis_name="s")
    per_w = N_PAIRS // 32
    return pl.kernel(
        _sc_gather_body,
        out_type=jax.ShapeDtypeStruct((N_PAIRS, HIDDEN), jnp.float32),
        mesh=mesh,
        scratch_types=[
            pltpu.VMEM((per_w,), jnp.int32),
            pltpu.VMEM((CHUNK, HIDDEN), jnp.float32),
            pltpu.VMEM((CHUNK, HIDDEN), jnp.float32),
            pltpu.SemaphoreType.DMA,
            pltpu.SemaphoreType.DMA,
        ],
    )(tab, idx)


def _head_body(xd_ref, xp_ref, b1_ref, w2_ref, b2_ref, o_ref):
    h = jnp.tanh(xd_ref[...] + xp_ref[...] + b1_ref[...])
    logits = lax.dot_general(
        w2_ref[...], h,
        dimension_numbers=(((0,), (1,)), ((), ())),
        preferred_element_type=jnp.float32)
    logits += b2_ref[...]
    m = jnp.max(logits, axis=0, keepdims=True)
    lse = m + jnp.log(jnp.sum(jnp.exp(logits - m), axis=0, keepdims=True))
    o_ref[...] = logits - lse


def _tc_head(xd, xp, b1, w2, b2):
    blk = 4096
    return pl.pallas_call(
        _head_body,
        grid=(N_PAIRS // blk,),
        in_specs=[
            pl.BlockSpec((blk, HIDDEN), lambda i: (i, 0)),
            pl.BlockSpec((blk, HIDDEN), lambda i: (i, 0)),
            pl.BlockSpec((1, HIDDEN), lambda i: (0, 0)),
            pl.BlockSpec((HIDDEN, 2), lambda i: (0, 0)),
            pl.BlockSpec((2, 1), lambda i: (0, 0)),
        ],
        out_specs=pl.BlockSpec((2, blk), lambda i: (0, i)),
        out_shape=jax.ShapeDtypeStruct((2, N_PAIRS), jnp.float32),
        compiler_params=pltpu.CompilerParams(
            dimension_semantics=("arbitrary",),
        ),
    )(xd, xp, b1, w2, b2)


def kernel(graph, dataset_index, iftrain, d, p, W1, b1, W2, b2):
    ds_t = dataset_index.T.astype(jnp.int32)
    didx = ds_t[0]
    pidx = ds_t[1]
    dp = _tc_embed(d.T, W1[:FEAT])
    xd = _sc_gather(dp, didx)
    pp = _tc_embed(p.T, W1[FEAT:])
    xp = _sc_gather(pp, pidx)
    out_t = _tc_head(xd, xp, b1.reshape(1, HIDDEN), W2, b2.reshape(2, 1))
    return out_t.T


# speedup vs baseline: 7.4897x; 1.1098x over previous
import jax
import jax.numpy as jnp
from jax import lax
from jax.experimental import pallas as pl
from jax.experimental.pallas import tpu as pltpu
from jax.experimental.pallas import tpu_sc as plsc

N_PAIRS = 65536
N_NODES = 100000
FEAT = 320
HIDDEN = 128
CHUNK = 128
N_SEG = 2
SEG = N_PAIRS // N_SEG


def _embed_body(dt_ref, pt_ref, wa_ref, wb_ref, od_ref, op_ref):
    dn = (((0,), (0,)), ((), ()))
    od_ref[...] = lax.dot_general(dt_ref[...], wa_ref[...], dn,
                                  preferred_element_type=jnp.float32)
    op_ref[...] = lax.dot_general(pt_ref[...], wb_ref[...], dn,
                                  preferred_element_type=jnp.float32)


def _tc_embed(d_t, p_t, wa, wb):
    blk = 2048
    out = jax.ShapeDtypeStruct((N_NODES, HIDDEN), jnp.float32)
    return pl.pallas_call(
        _embed_body,
        grid=((N_NODES + blk - 1) // blk,),
        in_specs=[
            pl.BlockSpec((FEAT, blk), lambda i: (0, i)),
            pl.BlockSpec((FEAT, blk), lambda i: (0, i)),
            pl.BlockSpec((FEAT, HIDDEN), lambda i: (0, 0)),
            pl.BlockSpec((FEAT, HIDDEN), lambda i: (0, 0)),
        ],
        out_specs=[
            pl.BlockSpec((blk, HIDDEN), lambda i: (i, 0)),
            pl.BlockSpec((blk, HIDDEN), lambda i: (i, 0)),
        ],
        out_shape=[out, out],
        compiler_params=pltpu.CompilerParams(
            dimension_semantics=("arbitrary",),
        ),
    )(d_t, p_t, wa, wb)


def _sc_gather_body(dp_hbm, pp_hbm, didx_hbm, pidx_hbm, outd_hbm, outp_hbm,
                    idx_v, rows_d0, rows_d1, rows_p0, rows_p1,
                    sem_d0, sem_d1, sem_p0, sem_p1):
    nc = 2
    wid = lax.axis_index("s") * nc + lax.axis_index("c")
    per_w = SEG // 32
    n_chunks = per_w // CHUNK
    base = wid * per_w

    pltpu.sync_copy(didx_hbm.at[pl.ds(base, per_w)], idx_v.at[pl.ds(0, per_w)])
    pltpu.sync_copy(pidx_hbm.at[pl.ds(base, per_w)],
                    idx_v.at[pl.ds(per_w, per_w)])

    slots = ((rows_d0, rows_p0, sem_d0, sem_p0),
             (rows_d1, rows_p1, sem_d1, sem_p1))

    def start(c, slot):
        rd, rp, sd, sp = slots[slot]
        pltpu.async_copy(dp_hbm.at[idx_v.at[pl.ds(c * CHUNK, CHUNK)]], rd, sd)
        pltpu.async_copy(pp_hbm.at[idx_v.at[pl.ds(per_w + c * CHUNK, CHUNK)]],
                         rp, sp)

    def finish(c, slot):
        rd, rp, sd, sp = slots[slot]
        off = base + c * CHUNK
        pltpu.make_async_copy(dp_hbm.at[pl.ds(0, CHUNK)], rd, sd).wait()
        pltpu.sync_copy(rd, outd_hbm.at[pl.ds(off, CHUNK)])
        pltpu.make_async_copy(pp_hbm.at[pl.ds(0, CHUNK)], rp, sp).wait()
        pltpu.sync_copy(rp, outp_hbm.at[pl.ds(off, CHUNK)])

    start(0, 0)

    def body(j, carry):
        c0 = 2 * j
        start(c0 + 1, 1)
        finish(c0, 0)
        start(c0 + 2, 0)
        finish(c0 + 1, 1)
        return carry

    lax.fori_loop(0, n_chunks // 2 - 1, body, 0)
    c_last = n_chunks - 2
    start(c_last + 1, 1)
    finish(c_last, 0)
    finish(c_last + 1, 1)


def _sc_gather(dp, pp, didx, pidx):
    mesh = plsc.VectorSubcoreMesh(core_axis_name="c", subcore_axis_name="s")
    per_w = SEG // 32
    out = jax.ShapeDtypeStruct((SEG, HIDDEN), jnp.float32)
    return pl.kernel(
        _sc_gather_body,
        out_type=(out, out),
        mesh=mesh,
        scratch_types=[
            pltpu.VMEM((2 * per_w,), jnp.int32),
            pltpu.VMEM((CHUNK, HIDDEN), jnp.float32),
            pltpu.VMEM((CHUNK, HIDDEN), jnp.float32),
            pltpu.VMEM((CHUNK, HIDDEN), jnp.float32),
            pltpu.VMEM((CHUNK, HIDDEN), jnp.float32),
            pltpu.SemaphoreType.DMA,
            pltpu.SemaphoreType.DMA,
            pltpu.SemaphoreType.DMA,
            pltpu.SemaphoreType.DMA,
        ],
    )(dp, pp, didx, pidx)


def _head_body(xd_ref, xp_ref, b1_ref, w2_ref, b2_ref, o_ref):
    h = jnp.tanh(xd_ref[...] + xp_ref[...] + b1_ref[...])
    logits = lax.dot_general(
        w2_ref[...], h,
        dimension_numbers=(((0,), (1,)), ((), ())),
        preferred_element_type=jnp.float32)
    logits += b2_ref[...]
    m = jnp.max(logits, axis=0, keepdims=True)
    lse = m + jnp.log(jnp.sum(jnp.exp(logits - m), axis=0, keepdims=True))
    o_ref[...] = logits - lse


def _tc_head(xd, xp, b1, w2, b2):
    blk = 4096
    return pl.pallas_call(
        _head_body,
        grid=(SEG // blk,),
        in_specs=[
            pl.BlockSpec((blk, HIDDEN), lambda i: (i, 0)),
            pl.BlockSpec((blk, HIDDEN), lambda i: (i, 0)),
            pl.BlockSpec((1, HIDDEN), lambda i: (0, 0)),
            pl.BlockSpec((HIDDEN, 2), lambda i: (0, 0)),
            pl.BlockSpec((2, 1), lambda i: (0, 0)),
        ],
        out_specs=pl.BlockSpec((2, blk), lambda i: (0, i)),
        out_shape=jax.ShapeDtypeStruct((2, SEG), jnp.float32),
        compiler_params=pltpu.CompilerParams(
            dimension_semantics=("arbitrary",),
        ),
    )(xd, xp, b1, w2, b2)


def kernel(graph, dataset_index, iftrain, d, p, W1, b1, W2, b2):
    ds_t = dataset_index.T.astype(jnp.int32)
    didx = ds_t[0]
    pidx = ds_t[1]
    dp, pp = _tc_embed(d.T, p.T, W1[:FEAT], W1[FEAT:])
    b1r = b1.reshape(1, HIDDEN)
    b2r = b2.reshape(2, 1)
    outs = []
    for s in range(N_SEG):
        lo = s * SEG
        xd, xp = _sc_gather(dp, pp,
                            lax.slice(didx, (lo,), (lo + SEG,)),
                            lax.slice(pidx, (lo,), (lo + SEG,)))
        outs.append(_tc_head(xd, xp, b1r, W2, b2r))
    return jnp.concatenate(outs, axis=1).T
